# baseline (speedup 1.0000x reference)
import math
import functools

import jax
import jax.numpy as jnp
import numpy as np
from jax.experimental import pallas as pl
from jax.experimental.pallas import tpu as pltpu
from jax.experimental.pallas import tpu_sc as plsc

DIM = 64
MAX_LEN = 5000
SQRT_DIM = math.sqrt(DIM)

LANES = 16
WINDOW = 128


def _make_pe_2d(seq_len: int) -> np.ndarray:
    position = np.arange(0, MAX_LEN, dtype=np.float64)[:, None]
    div_term = np.exp(
        np.arange(0, DIM, 2, dtype=np.float64) * -(math.log(10000.0) / DIM)
    )
    pe = np.zeros((MAX_LEN, DIM), dtype=np.float64)
    pe[:, 0::2] = np.sin(position * div_term)
    pe[:, 1::2] = np.cos(position * div_term)
    return pe[:seq_len].astype(np.float32)


@functools.partial(jax.jit, static_argnames=("S", "B"))
def _embed_sc(idx_flat, W, pe, *, S, B):
    N = S * B
    n_steps = N // WINDOW
    s_per_win = WINDOW // B

    mesh = plsc.VectorSubcoreMesh(core_axis_name="core",
                                  subcore_axis_name="subcore")

    @pl.kernel(
        out_type=jax.ShapeDtypeStruct((N, DIM), jnp.float32),
        mesh=mesh,
        compiler_params=pltpu.CompilerParams(use_tc_tiling_on_sc=False),
    )
    def kernel_fn(W_hbm, i_hbm, pe_hbm, o_hbm):
        def body(i_vmem, pe_vmem, o_vmem):
            pltpu.sync_copy(W_hbm.at[i_vmem.at[0]], o_vmem)
            for h in range(s_per_win):
                pe_regs = [pe_vmem[h, pl.ds(q * LANES, LANES)]
                           for q in range(DIM // LANES)]

                @pl.loop(0, B)
                def _(r, h=h, pe_regs=pe_regs):
                    row = h * B + r
                    for q in range(DIM // LANES):
                        sl = pl.ds(q * LANES, LANES)
                        o_vmem[row, sl] = o_vmem[row, sl] * SQRT_DIM + pe_regs[q]

        pltpu.emit_pipeline(
            body,
            grid=(n_steps,),
            in_specs=[
                pl.BlockSpec((1, WINDOW), index_map=lambda i: (0, i)),
                pl.BlockSpec((s_per_win, DIM), index_map=lambda i: (i, 0)),
            ],
            out_specs=[
                pl.BlockSpec((WINDOW, DIM), index_map=lambda i: (i, 0)),
            ],
            core_axis_name=("core", "subcore"),
            dimension_semantics=(pltpu.PARALLEL,),
        )(i_hbm, pe_hbm, o_hbm)

    return kernel_fn(W, idx_flat, pe)


def kernel(input, W):
    S, B, _ = input.shape
    idx_flat = input[..., 0].reshape(1, S * B)
    pe = jnp.asarray(_make_pe_2d(S))
    out = _embed_sc(idx_flat, W, pe, S=S, B=B)
    return out.reshape(S, B, DIM)


# speedup vs baseline: 1.1533x; 1.1533x over previous
import math
import functools

import jax
import jax.numpy as jnp
import numpy as np
from jax import lax
from jax.experimental import pallas as pl
from jax.experimental.pallas import tpu as pltpu
from jax.experimental.pallas import tpu_sc as plsc

DIM = 64
MAX_LEN = 5000
SQRT_DIM = math.sqrt(DIM)

LANES = 16
NWORKERS = 32
STREAM_W = 128
CHUNK = 512
NBUF = 3
NVREG = DIM // LANES


def _make_pe_2d(seq_len: int) -> np.ndarray:
    position = np.arange(0, MAX_LEN, dtype=np.float64)[:, None]
    div_term = np.exp(
        np.arange(0, DIM, 2, dtype=np.float64) * -(math.log(10000.0) / DIM)
    )
    pe = np.zeros((MAX_LEN, DIM), dtype=np.float64)
    pe[:, 0::2] = np.sin(position * div_term)
    pe[:, 1::2] = np.cos(position * div_term)
    return pe[:seq_len].astype(np.float32)


@functools.partial(jax.jit, static_argnames=("S", "B"))
def _embed_sc(idx3, W, pe, *, S, B):
    N = S * B
    per_w = N // NWORKERS
    n_chunks = per_w // CHUNK
    streams_per_chunk = CHUNK // STREAM_W
    s_per_chunk = CHUNK // B
    s_per_w = per_w // B

    mesh = plsc.VectorSubcoreMesh(core_axis_name="core",
                                  subcore_axis_name="subcore")

    @pl.kernel(
        out_type=jax.ShapeDtypeStruct((N, DIM), jnp.float32),
        mesh=mesh,
        compiler_params=pltpu.CompilerParams(use_tc_tiling_on_sc=False),
        scratch_types=[
            pltpu.VMEM((per_w // STREAM_W, STREAM_W), jnp.int32),
            pltpu.VMEM((s_per_w, DIM), jnp.float32),
            pltpu.VMEM((NBUF, CHUNK, DIM), jnp.float32),
            pltpu.SemaphoreType.DMA,
            pltpu.SemaphoreType.DMA((NBUF,)),
            pltpu.SemaphoreType.DMA((NBUF,)),
        ],
    )
    def kernel_fn(W_hbm, i_hbm, pe_hbm, o_hbm,
                  idx_v, pe_v, buf_v, sem_in, sem_g, sem_s):
        wid = lax.axis_index("core") * 16 + lax.axis_index("subcore")
        row0 = wid * per_w

        c0 = pltpu.async_copy(i_hbm.at[wid], idx_v, sem_in)
        c1 = pltpu.async_copy(pe_hbm.at[pl.ds(wid * s_per_w, s_per_w)],
                              pe_v, sem_in)
        c0.wait()
        c1.wait()

        def fire_gather(j, b):
            for k in range(streams_per_chunk):
                pltpu.async_copy(
                    W_hbm.at[idx_v.at[j * streams_per_chunk + k]],
                    buf_v.at[b, pl.ds(k * STREAM_W, STREAM_W)],
                    sem_g.at[b],
                )

        def wait_gather(j, b):
            for k in range(streams_per_chunk):
                pltpu.make_async_copy(
                    W_hbm.at[idx_v.at[j * streams_per_chunk + k]],
                    buf_v.at[b, pl.ds(k * STREAM_W, STREAM_W)],
                    sem_g.at[b],
                ).wait()

        def compute(j, b):
            for g in range(s_per_chunk):
                pe_regs = [pe_v[j * s_per_chunk + g, pl.ds(q * LANES, LANES)]
                           for q in range(NVREG)]

                @pl.loop(0, B)
                def _(r, g=g, pe_regs=pe_regs):
                    row = g * B + r
                    for q in range(NVREG):
                        sl = pl.ds(q * LANES, LANES)
                        buf_v[b, row, sl] = (buf_v[b, row, sl] * SQRT_DIM
                                             + pe_regs[q])

        def scatter(j, b):
            return pltpu.async_copy(
                buf_v.at[b],
                o_hbm.at[pl.ds(row0 + j * CHUNK, CHUNK)],
                sem_s.at[b],
            )

        scatter_handles = [None] * NBUF
        fire_gather(0, 0)
        for j in range(n_chunks):
            b = j % NBUF
            wait_gather(j, b)
            if j + 1 < n_chunks:
                nb = (j + 1) % NBUF
                if scatter_handles[nb] is not None:
                    scatter_handles[nb].wait()
                    scatter_handles[nb] = None
                fire_gather(j + 1, nb)
            compute(j, b)
            scatter_handles[b] = scatter(j, b)
        for h in scatter_handles:
            if h is not None:
                h.wait()

    return kernel_fn(W, idx3, pe)


def kernel(input, W):
    S, B, _ = input.shape
    N = S * B
    idx3 = input[..., 0].reshape(NWORKERS, (N // NWORKERS) // STREAM_W, STREAM_W)
    pe = jnp.asarray(_make_pe_2d(S))
    out = _embed_sc(idx3, W, pe, S=S, B=B)
    return out.reshape(S, B, DIM)
